# baseline (speedup 1.0000x reference)
import functools

import jax
import jax.numpy as jnp
from jax import lax
from jax.experimental import pallas as pl
from jax.experimental.pallas import tpu as pltpu
from jax.experimental.pallas import tpu_sc as plsc

NC = 2
NS = 16
NW = NC * NS
LANES = 16


def _logz_body(pred_ref, out_ref):
    x = pred_ref[0]
    m = jnp.max(x, axis=0)
    s = jnp.sum(jnp.exp(x - m[None]), axis=0)
    out_ref[0] = m + jnp.log(s)


def _logz_call(pred, rh):
    B, Q, H, W = pred.shape
    n_j = H // rh
    grid = (B * n_j,)
    return pl.pallas_call(
        _logz_body,
        grid=grid,
        in_specs=[pl.BlockSpec((1, Q, rh, W),
                               lambda g: (g // n_j, 0, g % n_j, 0))],
        out_specs=pl.BlockSpec((1, rh, W), lambda g: (g // n_j, g % n_j, 0)),
        out_shape=jax.ShapeDtypeStruct((B, H, W), jnp.float32),
    )(pred)


def _sc1_body(Q, HW, K, chunk,
              pred_hbm, binned_hbm, knn_hbm, kw_hbm, w_hbm,
              out_hbm, ws_hbm,
              knn_v, kw_v, w_v, t_v, addr_v, kws_v, g_v, ws_v, acc_v, sem):
    cid = lax.axis_index("c")
    sid = lax.axis_index("s")
    wid = sid * NC + cid
    base = wid * chunk
    b = base // HW
    hw0 = base - b * HW
    pred_base = b * (Q * HW)

    pltpu.sync_copy(knn_hbm, knn_v)
    pltpu.sync_copy(kw_hbm, kw_v)
    pltpu.sync_copy(w_hbm, w_v)
    pltpu.sync_copy(binned_hbm.at[pl.ds(base, chunk)], t_v)

    lane = jnp.arange(LANES, dtype=jnp.int32)
    n_grp = chunk // LANES

    def phase1(i, carry):
        t16 = t_v[pl.ds(i * LANES, LANES)]
        w16 = plsc.load_gather(w_v, [t16])
        hw = hw0 + i * LANES + lane
        tk = t16 * K
        ws = jnp.zeros((LANES,), jnp.float32)
        for k in range(K):
            ck = plsc.load_gather(knn_v, [tk + k])
            kwk = w16 * plsc.load_gather(kw_v, [tk + k])
            g = i * K + k
            addr_v[pl.ds(g * LANES, LANES)] = pred_base + ck * HW + hw
            kws_v[pl.ds(g * LANES, LANES)] = kwk
            ws = ws + kwk
        ws_v[pl.ds(i * LANES, LANES)] = ws
        return carry

    lax.fori_loop(0, n_grp, phase1, 0)
    pltpu.sync_copy(ws_v, ws_hbm.at[pl.ds(base, chunk)])

    pltpu.async_copy(pred_hbm.at[addr_v], g_v, sem).wait()

    def phase2(i, carry):
        acc = jnp.zeros((LANES,), jnp.float32)
        for k in range(K):
            g = i * K + k
            acc = acc + (kws_v[pl.ds(g * LANES, LANES)]
                         * g_v[pl.ds(g * LANES, LANES)])
        acc_v[...] = acc_v[...] + acc
        return carry

    acc_v[...] = jnp.zeros((LANES,), jnp.float32)
    lax.fori_loop(0, n_grp, phase2, 0)
    pltpu.sync_copy(acc_v, out_hbm.at[wid])


def _sc1_call(pred_flat, binned_flat, knn_flat, kw_flat, w_pad,
              Q, HW, K, chunk):
    N = binned_flat.shape[0]
    mesh = plsc.VectorSubcoreMesh(core_axis_name="c", subcore_axis_name="s")
    body = functools.partial(_sc1_body, Q, HW, K, chunk)
    return pl.kernel(
        body,
        out_type=(jax.ShapeDtypeStruct((NW, LANES), jnp.float32),
                  jax.ShapeDtypeStruct((N,), jnp.float32)),
        mesh=mesh,
        compiler_params=pltpu.CompilerParams(needs_layout_passes=False),
        scratch_types=[
            pltpu.VMEM((knn_flat.shape[0],), jnp.int32),
            pltpu.VMEM((kw_flat.shape[0],), jnp.float32),
            pltpu.VMEM((w_pad.shape[0],), jnp.float32),
            pltpu.VMEM((chunk,), jnp.int32),
            pltpu.VMEM((chunk * K,), jnp.int32),
            pltpu.VMEM((chunk * K,), jnp.float32),
            pltpu.VMEM((chunk * K,), jnp.float32),
            pltpu.VMEM((chunk,), jnp.float32),
            pltpu.VMEM((LANES,), jnp.float32),
            pltpu.SemaphoreType.DMA,
        ],
    )(pred_flat, binned_flat, knn_flat, kw_flat, w_pad)


def _sc2_body(chunk, ws_hbm, logz_hbm, out_hbm, ws_v, lz_v, acc_v):
    cid = lax.axis_index("c")
    sid = lax.axis_index("s")
    wid = sid * NC + cid
    base = wid * chunk
    pltpu.sync_copy(ws_hbm.at[pl.ds(base, chunk)], ws_v)
    pltpu.sync_copy(logz_hbm.at[pl.ds(base, chunk)], lz_v)

    def body(i, carry):
        acc_v[...] = acc_v[...] + (ws_v[pl.ds(i * LANES, LANES)]
                                   * lz_v[pl.ds(i * LANES, LANES)])
        return carry

    acc_v[...] = jnp.zeros((LANES,), jnp.float32)
    lax.fori_loop(0, chunk // LANES, body, 0)
    pltpu.sync_copy(acc_v, out_hbm.at[wid])


def _sc2_call(ws_flat, logz, chunk):
    mesh = plsc.VectorSubcoreMesh(core_axis_name="c", subcore_axis_name="s")
    body = functools.partial(_sc2_body, chunk)
    return pl.kernel(
        body,
        out_type=jax.ShapeDtypeStruct((NW, LANES), jnp.float32),
        mesh=mesh,
        compiler_params=pltpu.CompilerParams(needs_layout_passes=False),
        scratch_types=[
            pltpu.VMEM((chunk,), jnp.float32),
            pltpu.VMEM((chunk,), jnp.float32),
            pltpu.VMEM((LANES,), jnp.float32),
        ],
    )(ws_flat, logz)


def kernel(pred, _color, binned_color, knn_idx, knn_weights, weights):
    B, Q, H, W = pred.shape
    K = knn_idx.shape[1]
    HW = H * W
    N = B * HW
    chunk = N // NW

    pred_flat = pred.reshape(-1)
    binned_flat = binned_color.reshape(-1).astype(jnp.int32)
    knn_flat = jnp.pad(knn_idx.astype(jnp.int32).reshape(-1), (0, -(Q * K) % 8))
    kw_flat = jnp.pad(knn_weights.astype(jnp.float32).reshape(-1),
                      (0, -(Q * K) % 8))
    w_pad = jnp.pad(weights.astype(jnp.float32), (0, -Q % 8))

    part1, ws_flat = _sc1_call(pred_flat, binned_flat, knn_flat, kw_flat,
                               w_pad, Q, HW, K, chunk)
    logz = _logz_call(pred, rh=64).reshape(-1)
    part2 = _sc2_call(ws_flat, logz, chunk)
    return -(jnp.sum(part1) - jnp.sum(part2)) / N


# speedup vs baseline: 1.0243x; 1.0243x over previous
import functools

import jax
import jax.numpy as jnp
from jax import lax
from jax.experimental import pallas as pl
from jax.experimental.pallas import tpu as pltpu
from jax.experimental.pallas import tpu_sc as plsc

NC = 2
NS = 16
NW = NC * NS
LANES = 16


def _logz_body(pred_ref, out_ref):
    x = pred_ref[0]
    m = jnp.max(x, axis=0)
    s = jnp.sum(jnp.exp(x - m[None]), axis=0)
    out_ref[0] = m + jnp.log(s)


def _logz_call(pred, rh):
    B, Q, H, W = pred.shape
    n_j = H // rh
    grid = (B * n_j,)
    return pl.pallas_call(
        _logz_body,
        grid=grid,
        in_specs=[pl.BlockSpec((1, Q, rh, W),
                               lambda g: (g // n_j, 0, g % n_j, 0))],
        out_specs=pl.BlockSpec((1, rh, W), lambda g: (g // n_j, g % n_j, 0)),
        out_shape=jax.ShapeDtypeStruct((B, H, W), jnp.float32),
    )(pred)


def _sc1_body(Q, HW, K, chunk,
              pred_hbm, binned_hbm, knn_hbm, kw_hbm, w_hbm,
              out_hbm, ws_hbm,
              knn_v, kw_v, w_v, t_v, addr_v, kws_v, g_v, ws_v, acc_v, sem):
    cid = lax.axis_index("c")
    sid = lax.axis_index("s")
    wid = sid * NC + cid
    base = wid * chunk
    b = base // HW
    hw0 = base - b * HW
    pred_base = b * (Q * HW)

    pltpu.sync_copy(knn_hbm, knn_v)
    pltpu.sync_copy(kw_hbm, kw_v)
    pltpu.sync_copy(w_hbm, w_v)
    pltpu.sync_copy(binned_hbm.at[pl.ds(base, chunk)], t_v)

    lane = jnp.arange(LANES, dtype=jnp.int32)
    n_grp = chunk // LANES

    def phase1(i, carry):
        t16 = t_v[pl.ds(i * LANES, LANES)]
        w16 = plsc.load_gather(w_v, [t16])
        hw = hw0 + i * LANES + lane
        tk = t16 * K
        ws = jnp.zeros((LANES,), jnp.float32)
        for k in range(K):
            ck = plsc.load_gather(knn_v, [tk + k])
            kwk = w16 * plsc.load_gather(kw_v, [tk + k])
            g = i * K + k
            addr_v[pl.ds(g * LANES, LANES)] = pred_base + ck * HW + hw
            kws_v[pl.ds(g * LANES, LANES)] = kwk
            ws = ws + kwk
        ws_v[pl.ds(i * LANES, LANES)] = ws
        return carry

    def phase2(i, carry):
        acc = jnp.zeros((LANES,), jnp.float32)
        for k in range(K):
            g = i * K + k
            acc = acc + (kws_v[pl.ds(g * LANES, LANES)]
                         * g_v[pl.ds(g * LANES, LANES)])
        acc_v[...] = acc_v[...] + acc
        return carry

    NB = 4
    gpb = n_grp // NB
    epb = gpb * LANES * K
    acc_v[...] = jnp.zeros((LANES,), jnp.float32)
    descs = []
    for j in range(NB):
        lax.fori_loop(j * gpb, (j + 1) * gpb, phase1, 0)
        descs.append(pltpu.async_copy(
            pred_hbm.at[addr_v.at[pl.ds(j * epb, epb)]],
            g_v.at[pl.ds(j * epb, epb)], sem))
        if j >= 1:
            descs[j - 1].wait()
            lax.fori_loop((j - 1) * gpb, j * gpb, phase2, 0)
    pltpu.sync_copy(ws_v, ws_hbm.at[pl.ds(base, chunk)])
    descs[NB - 1].wait()
    lax.fori_loop((NB - 1) * gpb, NB * gpb, phase2, 0)
    pltpu.sync_copy(acc_v, out_hbm.at[wid])


def _sc1_call(pred_flat, binned_flat, knn_flat, kw_flat, w_pad,
              Q, HW, K, chunk):
    N = binned_flat.shape[0]
    mesh = plsc.VectorSubcoreMesh(core_axis_name="c", subcore_axis_name="s")
    body = functools.partial(_sc1_body, Q, HW, K, chunk)
    return pl.kernel(
        body,
        out_type=(jax.ShapeDtypeStruct((NW, LANES), jnp.float32),
                  jax.ShapeDtypeStruct((N,), jnp.float32)),
        mesh=mesh,
        compiler_params=pltpu.CompilerParams(needs_layout_passes=False),
        scratch_types=[
            pltpu.VMEM((knn_flat.shape[0],), jnp.int32),
            pltpu.VMEM((kw_flat.shape[0],), jnp.float32),
            pltpu.VMEM((w_pad.shape[0],), jnp.float32),
            pltpu.VMEM((chunk,), jnp.int32),
            pltpu.VMEM((chunk * K,), jnp.int32),
            pltpu.VMEM((chunk * K,), jnp.float32),
            pltpu.VMEM((chunk * K,), jnp.float32),
            pltpu.VMEM((chunk,), jnp.float32),
            pltpu.VMEM((LANES,), jnp.float32),
            pltpu.SemaphoreType.DMA,
        ],
    )(pred_flat, binned_flat, knn_flat, kw_flat, w_pad)


def _sc2_body(chunk, ws_hbm, logz_hbm, out_hbm, ws_v, lz_v, acc_v):
    cid = lax.axis_index("c")
    sid = lax.axis_index("s")
    wid = sid * NC + cid
    base = wid * chunk
    pltpu.sync_copy(ws_hbm.at[pl.ds(base, chunk)], ws_v)
    pltpu.sync_copy(logz_hbm.at[pl.ds(base, chunk)], lz_v)

    def body(i, carry):
        acc_v[...] = acc_v[...] + (ws_v[pl.ds(i * LANES, LANES)]
                                   * lz_v[pl.ds(i * LANES, LANES)])
        return carry

    acc_v[...] = jnp.zeros((LANES,), jnp.float32)
    lax.fori_loop(0, chunk // LANES, body, 0)
    pltpu.sync_copy(acc_v, out_hbm.at[wid])


def _sc2_call(ws_flat, logz, chunk):
    mesh = plsc.VectorSubcoreMesh(core_axis_name="c", subcore_axis_name="s")
    body = functools.partial(_sc2_body, chunk)
    return pl.kernel(
        body,
        out_type=jax.ShapeDtypeStruct((NW, LANES), jnp.float32),
        mesh=mesh,
        compiler_params=pltpu.CompilerParams(needs_layout_passes=False),
        scratch_types=[
            pltpu.VMEM((chunk,), jnp.float32),
            pltpu.VMEM((chunk,), jnp.float32),
            pltpu.VMEM((LANES,), jnp.float32),
        ],
    )(ws_flat, logz)


def kernel(pred, _color, binned_color, knn_idx, knn_weights, weights):
    B, Q, H, W = pred.shape
    K = knn_idx.shape[1]
    HW = H * W
    N = B * HW
    chunk = N // NW

    pred_flat = pred.reshape(-1)
    binned_flat = binned_color.reshape(-1).astype(jnp.int32)
    knn_flat = jnp.pad(knn_idx.astype(jnp.int32).reshape(-1), (0, -(Q * K) % 8))
    kw_flat = jnp.pad(knn_weights.astype(jnp.float32).reshape(-1),
                      (0, -(Q * K) % 8))
    w_pad = jnp.pad(weights.astype(jnp.float32), (0, -Q % 8))

    part1, ws_flat = _sc1_call(pred_flat, binned_flat, knn_flat, kw_flat,
                               w_pad, Q, HW, K, chunk)
    logz = _logz_call(pred, rh=128).reshape(-1)
    part2 = _sc2_call(ws_flat, logz, chunk)
    return -(jnp.sum(part1) - jnp.sum(part2)) / N
